# baseline (speedup 1.0000x reference)
import dataclasses

import jax
import jax.numpy as jnp
from jax import lax
from jax.experimental import pallas as pl
from jax.experimental.pallas import tpu as pltpu
from jax.experimental.pallas import tpu_sc as plsc

N, E, D, H, G = 10000, 320000, 128, 128, 64
HD = D // 2
NC, NS = 2, 16
NW = NC * NS
EPW = E // NW
WIN = 125
NWIN = 80
NPAD = 10112
RPS = NPAD // NS
DCH = NPAD // NS
F32 = jnp.float32

_MESH = plsc.VectorSubcoreMesh(core_axis_name="c", subcore_axis_name="s")

_SC_PARAMS = pltpu.CompilerParams()
if "needs_layout_passes" in pltpu.CompilerParams.__dataclass_fields__:
    _SC_PARAMS = dataclasses.replace(_SC_PARAMS, needs_layout_passes=False)
_SC_PARAMS_LINEAR = dataclasses.replace(_SC_PARAMS, use_tc_tiling_on_sc=False)


def _sc_degree_body(dst_hbm, deg0_hbm, deg1_hbm, didx, dpart, tmp, accd,
                    parts, sem):
    cid = lax.axis_index("c")
    sid = lax.axis_index("s")
    gw = cid * NS + sid

    @pl.loop(0, NPAD, step=16)
    def _(i):
        dpart[pl.ds(i, 16)] = jnp.zeros((16,), F32)

    pltpu.async_copy(dst_hbm.at[pl.ds(gw * EPW, EPW)], didx, sem).wait()
    ones = jnp.full((16,), 1.0, F32)

    @pl.loop(0, EPW, step=16)
    def _(i):
        plsc.addupdate_scatter(dpart, [didx[pl.ds(i, 16)]], ones)

    pltpu.sync_copy(dpart, parts.at[pl.ds(pl.multiple_of(sid * NPAD, 8),
                                          NPAD)])
    plsc.subcore_barrier()

    @pl.loop(0, DCH, step=16)
    def _(i):
        accd[pl.ds(i, 16)] = jnp.zeros((16,), F32)

    for r in range(NS):
        off = pl.multiple_of(r * NPAD + sid * DCH, 8)
        pltpu.sync_copy(parts.at[pl.ds(off, DCH)], tmp)

        @pl.loop(0, DCH, step=16)
        def _(i):
            accd[pl.ds(i, 16)] = accd[pl.ds(i, 16)] + tmp[pl.ds(i, 16)]

    doff = pl.multiple_of(sid * DCH, 8)

    @pl.when(cid == 0)
    def _():
        pltpu.sync_copy(accd, deg0_hbm.at[pl.ds(doff, DCH)])

    @pl.when(cid == 1)
    def _():
        pltpu.sync_copy(accd, deg1_hbm.at[pl.ds(doff, DCH)])


_sc_degree = pl.kernel(
    _sc_degree_body,
    out_type=(jax.ShapeDtypeStruct((NPAD,), F32),
              jax.ShapeDtypeStruct((NPAD,), F32)),
    mesh=_MESH,
    scratch_types=[
        pltpu.VMEM((EPW,), jnp.int32),
        pltpu.VMEM((NPAD,), F32),
        pltpu.VMEM((DCH,), F32),
        pltpu.VMEM((DCH,), F32),
        pltpu.VMEM_SHARED((NS * NPAD,), F32),
        pltpu.SemaphoreType.DMA,
    ],
    compiler_params=_SC_PARAMS,
)


def _sc_scatter_body(yl_hbm, yr_hbm, src3_hbm, dst3_hbm, zeros_hbm,
                     ol_hbm, or_hbm, sidx, didx, rows_a, rows_b, rows_c,
                     rows_d, acc, semi, sema, semb, semc, semd):
    cid = lax.axis_index("c")
    sid = lax.axis_index("s")
    gw = cid * NS + sid
    row0 = pl.multiple_of(sid * RPS, 8)

    pltpu.async_copy(src3_hbm.at[gw], sidx, semi).wait()
    pltpu.async_copy(dst3_hbm.at[gw], didx, semi).wait()

    def _drain(buf, sem):
        pltpu.make_async_copy(yl_hbm.at[pl.ds(0, WIN)], buf, sem).wait()

    for y_hbm, o_hbm in ((yl_hbm, ol_hbm), (yr_hbm, or_hbm)):
        pltpu.sync_copy(zeros_hbm.at[pl.ds(row0, RPS)],
                        acc.at[pl.ds(row0, RPS)])
        plsc.subcore_barrier()

        pltpu.async_copy(y_hbm.at[sidx.at[0]], rows_a, sema)
        pltpu.async_copy(y_hbm.at[sidx.at[1]], rows_b, semb)
        pltpu.async_copy(y_hbm.at[sidx.at[2]], rows_c, semc)

        @pl.loop(0, NWIN, step=4)
        def _(j):
            pltpu.async_copy(y_hbm.at[sidx.at[j + 3]], rows_d, semd)
            _drain(rows_a, sema)
            pltpu.sync_copy(rows_a, acc.at[didx.at[j]], add=True)

            @pl.when(j + 4 < NWIN)
            def _():
                pltpu.async_copy(y_hbm.at[sidx.at[j + 4]], rows_a, sema)

            _drain(rows_b, semb)
            pltpu.sync_copy(rows_b, acc.at[didx.at[j + 1]], add=True)

            @pl.when(j + 5 < NWIN)
            def _():
                pltpu.async_copy(y_hbm.at[sidx.at[j + 5]], rows_b, semb)

            _drain(rows_c, semc)
            pltpu.sync_copy(rows_c, acc.at[didx.at[j + 2]], add=True)

            @pl.when(j + 6 < NWIN)
            def _():
                pltpu.async_copy(y_hbm.at[sidx.at[j + 6]], rows_c, semc)

            _drain(rows_d, semd)
            pltpu.sync_copy(rows_d, acc.at[didx.at[j + 3]], add=True)

        plsc.subcore_barrier()
        pltpu.sync_copy(acc.at[pl.ds(row0, RPS)],
                        o_hbm.at[cid, pl.ds(row0, RPS)])
        plsc.subcore_barrier()


_sc_scatter = pl.kernel(
    _sc_scatter_body,
    out_type=(jax.ShapeDtypeStruct((NC, NPAD, HD), F32),
              jax.ShapeDtypeStruct((NC, NPAD, HD), F32)),
    mesh=_MESH,
    scratch_types=[
        pltpu.VMEM((NWIN, WIN), jnp.int32),
        pltpu.VMEM((NWIN, WIN), jnp.int32),
        pltpu.VMEM((WIN, HD), F32),
        pltpu.VMEM((WIN, HD), F32),
        pltpu.VMEM((WIN, HD), F32),
        pltpu.VMEM((WIN, HD), F32),
        pltpu.VMEM_SHARED((NPAD, HD), F32),
        pltpu.SemaphoreType.DMA,
        pltpu.SemaphoreType.DMA,
        pltpu.SemaphoreType.DMA,
        pltpu.SemaphoreType.DMA,
        pltpu.SemaphoreType.DMA,
    ],
    compiler_params=_SC_PARAMS_LINEAR,
)


BLK = 1000
HIGH = lax.Precision.HIGHEST


def _tc_matmul_body(x_ref, w_ref, o_ref):
    o_ref[...] = jnp.dot(x_ref[...], w_ref[...], preferred_element_type=F32,
                         precision=HIGH)


_tc_matmul = pl.pallas_call(
    _tc_matmul_body,
    grid=(N // BLK,),
    in_specs=[pl.BlockSpec((BLK, D), lambda i: (i, 0)),
              pl.BlockSpec((D, H), lambda i: (0, 0))],
    out_specs=pl.BlockSpec((BLK, H), lambda i: (i, 0)),
    out_shape=jax.ShapeDtypeStruct((N, H), F32),
)


def _tc_scale_body(d0_ref, d1_ref, xw_ref, yl_ref, yr_ref, dinv_ref):
    deg = d0_ref[...] + d1_ref[...] + 1.0
    dinv = lax.rsqrt(deg)
    dinv_ref[...] = dinv
    y = xw_ref[...] * dinv
    yl_ref[...] = y[:, :HD]
    yr_ref[...] = y[:, HD:]


_tc_scale = pl.pallas_call(
    _tc_scale_body,
    grid=(N // BLK,),
    in_specs=[pl.BlockSpec((BLK, 1), lambda i: (i, 0)),
              pl.BlockSpec((BLK, 1), lambda i: (i, 0)),
              pl.BlockSpec((BLK, H), lambda i: (i, 0))],
    out_specs=[pl.BlockSpec((BLK, HD), lambda i: (i, 0)),
               pl.BlockSpec((BLK, HD), lambda i: (i, 0)),
               pl.BlockSpec((BLK, 1), lambda i: (i, 0))],
    out_shape=[jax.ShapeDtypeStruct((NPAD, HD), F32),
               jax.ShapeDtypeStruct((NPAD, HD), F32),
               jax.ShapeDtypeStruct((N, 1), F32)],
)


def _tc_mid_body(al0_ref, al1_ref, ar0_ref, ar1_ref, yl_ref, yr_ref,
                 dinv_ref, b1_ref, w2_ref, ol_ref, or_ref):
    dinv = dinv_ref[...]
    left = al0_ref[0] + al1_ref[0] + yl_ref[...]
    right = ar0_ref[0] + ar1_ref[0] + yr_ref[...]
    t = jnp.concatenate([left, right], axis=1) * dinv + b1_ref[...]
    h = jnp.maximum(t, 0.0)
    y2 = jnp.dot(h, w2_ref[...], preferred_element_type=F32,
                 precision=HIGH) * dinv
    ol_ref[...] = y2[:, :HD]
    or_ref[...] = y2[:, HD:]


_tc_mid = pl.pallas_call(
    _tc_mid_body,
    grid=(N // BLK,),
    in_specs=[pl.BlockSpec((1, BLK, HD), lambda i: (0, i, 0)),
              pl.BlockSpec((1, BLK, HD), lambda i: (1, i, 0)),
              pl.BlockSpec((1, BLK, HD), lambda i: (0, i, 0)),
              pl.BlockSpec((1, BLK, HD), lambda i: (1, i, 0)),
              pl.BlockSpec((BLK, HD), lambda i: (i, 0)),
              pl.BlockSpec((BLK, HD), lambda i: (i, 0)),
              pl.BlockSpec((BLK, 1), lambda i: (i, 0)),
              pl.BlockSpec((1, H), lambda i: (0, 0)),
              pl.BlockSpec((H, H), lambda i: (0, 0))],
    out_specs=[pl.BlockSpec((BLK, HD), lambda i: (i, 0)),
               pl.BlockSpec((BLK, HD), lambda i: (i, 0))],
    out_shape=[jax.ShapeDtypeStruct((NPAD, HD), F32),
               jax.ShapeDtypeStruct((NPAD, HD), F32)],
)


def _tc_head_body(cl0_ref, cl1_ref, cr0_ref, cr1_ref, yl_ref, yr_ref,
                  dinv_ref, b2_ref, batch_ref, o_ref, pooled):
    i = pl.program_id(0)

    @pl.when(i == 0)
    def _():
        pooled[...] = jnp.full((G, H), -jnp.inf, F32)

    left = cl0_ref[0] + cl1_ref[0] + yl_ref[...]
    right = cr0_ref[0] + cr1_ref[0] + yr_ref[...]
    blk = jnp.concatenate([left, right], axis=1) * dinv_ref[...] \
        + b2_ref[...]
    bb = batch_ref[...]
    glo = jnp.min(bb)
    ghi = jnp.max(bb)

    def body(g, carry):
        v = jnp.where(bb == g, blk, -jnp.inf)
        red = jnp.max(v, axis=0, keepdims=True)
        pooled[pl.ds(g, 1), :] = jnp.maximum(pooled[pl.ds(g, 1), :], red)
        return carry

    lax.fori_loop(glo, ghi + 1, body, 0)

    @pl.when(i == N // BLK - 1)
    def _():
        p = pooled[...]
        mx = jnp.max(p, axis=1, keepdims=True)
        s = jnp.sum(jnp.exp(p - mx), axis=1, keepdims=True)
        o_ref[...] = p - mx - jnp.log(s)


_tc_head = pl.pallas_call(
    _tc_head_body,
    grid=(N // BLK,),
    in_specs=[pl.BlockSpec((1, BLK, HD), lambda i: (0, i, 0)),
              pl.BlockSpec((1, BLK, HD), lambda i: (1, i, 0)),
              pl.BlockSpec((1, BLK, HD), lambda i: (0, i, 0)),
              pl.BlockSpec((1, BLK, HD), lambda i: (1, i, 0)),
              pl.BlockSpec((BLK, HD), lambda i: (i, 0)),
              pl.BlockSpec((BLK, HD), lambda i: (i, 0)),
              pl.BlockSpec((BLK, 1), lambda i: (i, 0)),
              pl.BlockSpec((1, H), lambda i: (0, 0)),
              pl.BlockSpec((BLK, 1), lambda i: (i, 0))],
    out_specs=pl.BlockSpec((G, H), lambda i: (0, 0)),
    out_shape=jax.ShapeDtypeStruct((G, H), F32),
    scratch_shapes=[pltpu.VMEM((G, H), F32)],
)


def kernel(x, edge_index, batch, W1, b1, W2, b2):
    ei = edge_index.astype(jnp.int32)
    dst = ei[1]
    src3 = ei[0].reshape(NW, NWIN, WIN)
    dst3 = dst.reshape(NW, NWIN, WIN)
    zeros = jnp.zeros((NPAD, HD), F32)

    deg0, deg1 = _sc_degree(dst)
    xw1 = _tc_matmul(x, W1)
    yl1, yr1, dinv = _tc_scale(deg0.reshape(NPAD, 1), deg1.reshape(NPAD, 1),
                               xw1)
    al, ar = _sc_scatter(yl1, yr1, src3, dst3, zeros)
    y2l, y2r = _tc_mid(al, al, ar, ar, yl1, yr1, dinv, b1.reshape(1, H), W2)
    cl, cr = _sc_scatter(y2l, y2r, src3, dst3, zeros)
    return _tc_head(cl, cl, cr, cr, y2l, y2r, dinv, b2.reshape(1, H),
                    batch.astype(jnp.int32).reshape(N, 1))


# speedup vs baseline: 1.0093x; 1.0093x over previous
import dataclasses

import jax
import jax.numpy as jnp
from jax import lax
from jax.experimental import pallas as pl
from jax.experimental.pallas import tpu as pltpu
from jax.experimental.pallas import tpu_sc as plsc

N, E, D, H, G = 10000, 320000, 128, 128, 64
HD = D // 2
NC, NS = 2, 16
NW = NC * NS
EPW = E // NW
WIN = 125
NWIN = 80
NPAD = 10112
RPS = NPAD // NS
DCH = NPAD // NS
F32 = jnp.float32

_MESH = plsc.VectorSubcoreMesh(core_axis_name="c", subcore_axis_name="s")

_SC_PARAMS = pltpu.CompilerParams()
if "needs_layout_passes" in pltpu.CompilerParams.__dataclass_fields__:
    _SC_PARAMS = dataclasses.replace(_SC_PARAMS, needs_layout_passes=False)
_SC_PARAMS_LINEAR = dataclasses.replace(_SC_PARAMS, use_tc_tiling_on_sc=False)


def _sc_degree_body(dst_hbm, deg0_hbm, deg1_hbm, didx, dpart, tmp, accd,
                    parts, sem):
    cid = lax.axis_index("c")
    sid = lax.axis_index("s")
    gw = cid * NS + sid

    @pl.loop(0, NPAD, step=16)
    def _(i):
        dpart[pl.ds(i, 16)] = jnp.zeros((16,), F32)

    pltpu.async_copy(dst_hbm.at[pl.ds(gw * EPW, EPW)], didx, sem).wait()
    ones = jnp.full((16,), 1.0, F32)

    @pl.loop(0, EPW, step=16)
    def _(i):
        plsc.addupdate_scatter(dpart, [didx[pl.ds(i, 16)]], ones)

    pltpu.sync_copy(dpart, parts.at[pl.ds(pl.multiple_of(sid * NPAD, 8),
                                          NPAD)])
    plsc.subcore_barrier()

    @pl.loop(0, DCH, step=16)
    def _(i):
        accd[pl.ds(i, 16)] = jnp.zeros((16,), F32)

    for r in range(NS):
        off = pl.multiple_of(r * NPAD + sid * DCH, 8)
        pltpu.sync_copy(parts.at[pl.ds(off, DCH)], tmp)

        @pl.loop(0, DCH, step=16)
        def _(i):
            accd[pl.ds(i, 16)] = accd[pl.ds(i, 16)] + tmp[pl.ds(i, 16)]

    doff = pl.multiple_of(sid * DCH, 8)

    @pl.when(cid == 0)
    def _():
        pltpu.sync_copy(accd, deg0_hbm.at[pl.ds(doff, DCH)])

    @pl.when(cid == 1)
    def _():
        pltpu.sync_copy(accd, deg1_hbm.at[pl.ds(doff, DCH)])


_sc_degree = pl.kernel(
    _sc_degree_body,
    out_type=(jax.ShapeDtypeStruct((NPAD,), F32),
              jax.ShapeDtypeStruct((NPAD,), F32)),
    mesh=_MESH,
    scratch_types=[
        pltpu.VMEM((EPW,), jnp.int32),
        pltpu.VMEM((NPAD,), F32),
        pltpu.VMEM((DCH,), F32),
        pltpu.VMEM((DCH,), F32),
        pltpu.VMEM_SHARED((NS * NPAD,), F32),
        pltpu.SemaphoreType.DMA,
    ],
    compiler_params=_SC_PARAMS,
)


def _sc_scatter_body(yl_hbm, yr_hbm, src3_hbm, dst3_hbm, zeros_hbm,
                     ol_hbm, or_hbm, sidx, didx,
                     r0, r1, r2, r3, r4, r5, r6, r7, acc,
                     semi, s0, s1, s2, s3, s4, s5, s6, s7):
    cid = lax.axis_index("c")
    sid = lax.axis_index("s")
    gw = cid * NS + sid
    row0 = pl.multiple_of(sid * RPS, 8)
    bufs = (r0, r1, r2, r3, r4, r5, r6, r7)
    sems = (s0, s1, s2, s3, s4, s5, s6, s7)

    pltpu.async_copy(src3_hbm.at[gw], sidx, semi).wait()
    pltpu.async_copy(dst3_hbm.at[gw], didx, semi).wait()

    def _drain(buf, sem):
        pltpu.make_async_copy(yl_hbm.at[pl.ds(0, WIN)], buf, sem).wait()

    for y_hbm, o_hbm in ((yl_hbm, ol_hbm), (yr_hbm, or_hbm)):
        pltpu.sync_copy(zeros_hbm.at[pl.ds(row0, RPS)],
                        acc.at[pl.ds(row0, RPS)])
        plsc.subcore_barrier()

        for k in range(7):
            pltpu.async_copy(y_hbm.at[sidx.at[k]], bufs[k], sems[k])

        @pl.loop(0, NWIN, step=8)
        def _(j):
            for k in range(8):
                w = j + k

                @pl.when(w + 7 < NWIN)
                def _():
                    pltpu.async_copy(y_hbm.at[sidx.at[w + 7]],
                                     bufs[(k + 7) % 8], sems[(k + 7) % 8])

                _drain(bufs[k], sems[k])
                pltpu.sync_copy(bufs[k], acc.at[didx.at[w]], add=True)

        plsc.subcore_barrier()
        pltpu.sync_copy(acc.at[pl.ds(row0, RPS)],
                        o_hbm.at[cid, pl.ds(row0, RPS)])
        plsc.subcore_barrier()


_sc_scatter = pl.kernel(
    _sc_scatter_body,
    out_type=(jax.ShapeDtypeStruct((NC, NPAD, HD), F32),
              jax.ShapeDtypeStruct((NC, NPAD, HD), F32)),
    mesh=_MESH,
    scratch_types=[
        pltpu.VMEM((NWIN, WIN), jnp.int32),
        pltpu.VMEM((NWIN, WIN), jnp.int32),
    ] + [pltpu.VMEM((WIN, HD), F32)] * 8 + [
        pltpu.VMEM_SHARED((NPAD, HD), F32),
    ] + [pltpu.SemaphoreType.DMA] * 9,
    compiler_params=_SC_PARAMS_LINEAR,
)


BLK = 1000
HIGH = lax.Precision.DEFAULT


def _tc_matmul_body(x_ref, w_ref, o_ref):
    o_ref[...] = jnp.dot(x_ref[...], w_ref[...], preferred_element_type=F32,
                         precision=HIGH)


_tc_matmul = pl.pallas_call(
    _tc_matmul_body,
    grid=(N // BLK,),
    in_specs=[pl.BlockSpec((BLK, D), lambda i: (i, 0)),
              pl.BlockSpec((D, H), lambda i: (0, 0))],
    out_specs=pl.BlockSpec((BLK, H), lambda i: (i, 0)),
    out_shape=jax.ShapeDtypeStruct((N, H), F32),
)


def _tc_scale_body(d0_ref, d1_ref, xw_ref, yl_ref, yr_ref, dinv_ref):
    deg = d0_ref[...] + d1_ref[...] + 1.0
    dinv = lax.rsqrt(deg)
    dinv_ref[...] = dinv
    y = xw_ref[...] * dinv
    yl_ref[...] = y[:, :HD]
    yr_ref[...] = y[:, HD:]


_tc_scale = pl.pallas_call(
    _tc_scale_body,
    grid=(N // BLK,),
    in_specs=[pl.BlockSpec((BLK, 1), lambda i: (i, 0)),
              pl.BlockSpec((BLK, 1), lambda i: (i, 0)),
              pl.BlockSpec((BLK, H), lambda i: (i, 0))],
    out_specs=[pl.BlockSpec((BLK, HD), lambda i: (i, 0)),
               pl.BlockSpec((BLK, HD), lambda i: (i, 0)),
               pl.BlockSpec((BLK, 1), lambda i: (i, 0))],
    out_shape=[jax.ShapeDtypeStruct((NPAD, HD), F32),
               jax.ShapeDtypeStruct((NPAD, HD), F32),
               jax.ShapeDtypeStruct((N, 1), F32)],
)


def _tc_mid_body(al0_ref, al1_ref, ar0_ref, ar1_ref, yl_ref, yr_ref,
                 dinv_ref, b1_ref, w2_ref, ol_ref, or_ref):
    dinv = dinv_ref[...]
    left = al0_ref[0] + al1_ref[0] + yl_ref[...]
    right = ar0_ref[0] + ar1_ref[0] + yr_ref[...]
    t = jnp.concatenate([left, right], axis=1) * dinv + b1_ref[...]
    h = jnp.maximum(t, 0.0)
    y2 = jnp.dot(h, w2_ref[...], preferred_element_type=F32,
                 precision=HIGH) * dinv
    ol_ref[...] = y2[:, :HD]
    or_ref[...] = y2[:, HD:]


_tc_mid = pl.pallas_call(
    _tc_mid_body,
    grid=(N // BLK,),
    in_specs=[pl.BlockSpec((1, BLK, HD), lambda i: (0, i, 0)),
              pl.BlockSpec((1, BLK, HD), lambda i: (1, i, 0)),
              pl.BlockSpec((1, BLK, HD), lambda i: (0, i, 0)),
              pl.BlockSpec((1, BLK, HD), lambda i: (1, i, 0)),
              pl.BlockSpec((BLK, HD), lambda i: (i, 0)),
              pl.BlockSpec((BLK, HD), lambda i: (i, 0)),
              pl.BlockSpec((BLK, 1), lambda i: (i, 0)),
              pl.BlockSpec((1, H), lambda i: (0, 0)),
              pl.BlockSpec((H, H), lambda i: (0, 0))],
    out_specs=[pl.BlockSpec((BLK, HD), lambda i: (i, 0)),
               pl.BlockSpec((BLK, HD), lambda i: (i, 0))],
    out_shape=[jax.ShapeDtypeStruct((NPAD, HD), F32),
               jax.ShapeDtypeStruct((NPAD, HD), F32)],
)


def _tc_head_body(cl0_ref, cl1_ref, cr0_ref, cr1_ref, yl_ref, yr_ref,
                  dinv_ref, b2_ref, batch_ref, o_ref, pooled):
    i = pl.program_id(0)

    @pl.when(i == 0)
    def _():
        pooled[...] = jnp.full((G, H), -jnp.inf, F32)

    left = cl0_ref[0] + cl1_ref[0] + yl_ref[...]
    right = cr0_ref[0] + cr1_ref[0] + yr_ref[...]
    blk = jnp.concatenate([left, right], axis=1) * dinv_ref[...] \
        + b2_ref[...]
    bb = batch_ref[...]
    glo = jnp.min(bb)
    ghi = jnp.max(bb)

    def body(g, carry):
        v = jnp.where(bb == g, blk, -jnp.inf)
        red = jnp.max(v, axis=0, keepdims=True)
        pooled[pl.ds(g, 1), :] = jnp.maximum(pooled[pl.ds(g, 1), :], red)
        return carry

    lax.fori_loop(glo, ghi + 1, body, 0)

    @pl.when(i == N // BLK - 1)
    def _():
        p = pooled[...]
        mx = jnp.max(p, axis=1, keepdims=True)
        s = jnp.sum(jnp.exp(p - mx), axis=1, keepdims=True)
        o_ref[...] = p - mx - jnp.log(s)


_tc_head = pl.pallas_call(
    _tc_head_body,
    grid=(N // BLK,),
    in_specs=[pl.BlockSpec((1, BLK, HD), lambda i: (0, i, 0)),
              pl.BlockSpec((1, BLK, HD), lambda i: (1, i, 0)),
              pl.BlockSpec((1, BLK, HD), lambda i: (0, i, 0)),
              pl.BlockSpec((1, BLK, HD), lambda i: (1, i, 0)),
              pl.BlockSpec((BLK, HD), lambda i: (i, 0)),
              pl.BlockSpec((BLK, HD), lambda i: (i, 0)),
              pl.BlockSpec((BLK, 1), lambda i: (i, 0)),
              pl.BlockSpec((1, H), lambda i: (0, 0)),
              pl.BlockSpec((BLK, 1), lambda i: (i, 0))],
    out_specs=pl.BlockSpec((G, H), lambda i: (0, 0)),
    out_shape=jax.ShapeDtypeStruct((G, H), F32),
    scratch_shapes=[pltpu.VMEM((G, H), F32)],
)


def kernel(x, edge_index, batch, W1, b1, W2, b2):
    ei = edge_index.astype(jnp.int32)
    dst = ei[1]
    src3 = ei[0].reshape(NW, NWIN, WIN)
    dst3 = dst.reshape(NW, NWIN, WIN)
    zeros = jnp.zeros((NPAD, HD), F32)

    deg0, deg1 = _sc_degree(dst)
    xw1 = _tc_matmul(x, W1)
    yl1, yr1, dinv = _tc_scale(deg0.reshape(NPAD, 1), deg1.reshape(NPAD, 1),
                               xw1)
    al, ar = _sc_scatter(yl1, yr1, src3, dst3, zeros)
    y2l, y2r = _tc_mid(al, al, ar, ar, yl1, yr1, dinv, b1.reshape(1, H), W2)
    cl, cr = _sc_scatter(y2l, y2r, src3, dst3, zeros)
    return _tc_head(cl, cl, cr, cr, y2l, y2r, dinv, b2.reshape(1, H),
                    batch.astype(jnp.int32).reshape(N, 1))
